# Initial kernel scaffold; baseline (speedup 1.0000x reference)
import jax
import jax.numpy as jnp
from jax.experimental import pallas as pl


def kernel(Tokens, table):
    raise NotImplementedError("write your pallas kernel here")


# speedup vs baseline: 7.4156x; 7.4156x over previous
import functools
import math

import jax
import jax.numpy as jnp
from jax import lax
from jax.experimental import pallas as pl
from jax.experimental.pallas import tpu as pltpu
from jax.experimental.pallas import tpu_sc as plsc

_D = 128
_SCALE = math.sqrt(float(_D))


def _scale_body(x_ref, o_ref):
    o_ref[...] = x_ref[...] * _SCALE


@functools.lru_cache(maxsize=None)
def _make_scale(V, D):
    blk = 2000
    assert V % blk == 0
    return pl.pallas_call(
        _scale_body,
        out_shape=jax.ShapeDtypeStruct((V, D), jnp.float32),
        grid=(V // blk,),
        in_specs=[pl.BlockSpec((blk, D), lambda i: (i, 0))],
        out_specs=pl.BlockSpec((blk, D), lambda i: (i, 0)),
    )


@functools.lru_cache(maxsize=None)
def _make_gather(V, D, B):
    info = plsc.get_sparse_core_info()
    NC, NS = info.num_cores, info.num_subcores
    NW = NC * NS
    C = 128
    assert B % (NW * C) == 0
    b_per_w = B // NW
    n_chunks = b_per_w // C
    mesh = plsc.VectorSubcoreMesh(core_axis_name="c", subcore_axis_name="s")

    @functools.partial(
        pl.kernel,
        out_type=jax.ShapeDtypeStruct((B, D), jnp.float32),
        mesh=mesh,
        scratch_types=[
            pltpu.VMEM((n_chunks, C), jnp.int32),
            pltpu.VMEM((C, D), jnp.float32),
            pltpu.VMEM((C, D), jnp.float32),
            pltpu.SemaphoreType.DMA,
            pltpu.SemaphoreType.DMA,
            pltpu.SemaphoreType.DMA,
            pltpu.SemaphoreType.DMA,
        ],
    )
    def gather_kernel(idx_hbm, table_hbm, out_hbm,
                      idx_v, rows0, rows1, g0, g1, s0, s1):
        wid = lax.axis_index("s") * NC + lax.axis_index("c")
        base = wid * b_per_w
        rows = (rows0, rows1)
        gsem = (g0, g1)
        ssem = (s0, s1)

        pltpu.sync_copy(idx_hbm.at[pl.ds(wid * n_chunks, n_chunks)], idx_v)

        def gather_start(i, b):
            pltpu.async_copy(table_hbm.at[idx_v.at[i]], rows[b], gsem[b])

        def gather_wait(i, b):
            pltpu.make_async_copy(
                table_hbm.at[idx_v.at[i]], rows[b], gsem[b]).wait()

        def scatter_start(i, b):
            pltpu.async_copy(
                rows[b], out_hbm.at[pl.ds(base + i * C, C)], ssem[b])

        def scatter_wait(i, b):
            pltpu.make_async_copy(
                rows[b], out_hbm.at[pl.ds(base + i * C, C)], ssem[b]).wait()

        gather_start(0, 0)
        gather_start(1, 1)
        gather_wait(0, 0)
        scatter_start(0, 0)
        gather_wait(1, 1)
        scatter_start(1, 1)

        def body(j, carry):
            i0 = 2 * j
            i1 = i0 + 1
            scatter_wait(i0 - 2, 0)
            gather_start(i0, 0)
            scatter_wait(i1 - 2, 1)
            gather_start(i1, 1)
            gather_wait(i0, 0)
            scatter_start(i0, 0)
            gather_wait(i1, 1)
            scatter_start(i1, 1)
            return carry

        lax.fori_loop(1, n_chunks // 2, body, 0)

        scatter_wait(n_chunks - 2, 0)
        scatter_wait(n_chunks - 1, 1)

    return gather_kernel


def kernel(Tokens, table):
    S, T = Tokens.shape
    V, D = table.shape
    B = S * T
    scaled = _make_scale(V, D)(table)
    idx2d = Tokens.reshape(B // 128, 128).astype(jnp.int32)
    out = _make_gather(V, D, B)(idx2d, scaled)
    return out.reshape(S, T, D)


# speedup vs baseline: 7.8083x; 1.0530x over previous
import functools
import math

import jax
import jax.numpy as jnp
from jax import lax
from jax.experimental import pallas as pl
from jax.experimental.pallas import tpu as pltpu
from jax.experimental.pallas import tpu_sc as plsc

_D = 128
_SCALE = math.sqrt(float(_D))


def _scale_body(x_ref, o_ref):
    o_ref[...] = x_ref[...] * _SCALE


@functools.lru_cache(maxsize=None)
def _make_scale(V, D):
    blk = 2000
    assert V % blk == 0
    return pl.pallas_call(
        _scale_body,
        out_shape=jax.ShapeDtypeStruct((V, D), jnp.float32),
        grid=(V // blk,),
        in_specs=[pl.BlockSpec((blk, D), lambda i: (i, 0))],
        out_specs=pl.BlockSpec((blk, D), lambda i: (i, 0)),
    )


@functools.lru_cache(maxsize=None)
def _make_gather(V, D, B):
    info = plsc.get_sparse_core_info()
    NC, NS = info.num_cores, info.num_subcores
    NW = NC * NS
    C = 128
    G = 2
    CB = C * G
    assert B % (NW * CB) == 0
    b_per_w = B // NW
    n_idx = b_per_w // C
    n_chunks = b_per_w // CB
    mesh = plsc.VectorSubcoreMesh(core_axis_name="c", subcore_axis_name="s")

    @functools.partial(
        pl.kernel,
        out_type=jax.ShapeDtypeStruct((B, D), jnp.float32),
        mesh=mesh,
        scratch_types=[
            pltpu.VMEM((n_idx, C), jnp.int32),
            pltpu.VMEM((CB, D), jnp.float32),
            pltpu.VMEM((CB, D), jnp.float32),
            pltpu.SemaphoreType.DMA,
            pltpu.SemaphoreType.DMA,
            pltpu.SemaphoreType.DMA,
            pltpu.SemaphoreType.DMA,
        ],
    )
    def gather_kernel(idx_hbm, table_hbm, out_hbm,
                      idx_v, rows0, rows1, g0, g1, s0, s1):
        wid = lax.axis_index("s") * NC + lax.axis_index("c")
        base = wid * b_per_w
        rows = (rows0, rows1)
        gsem = (g0, g1)
        ssem = (s0, s1)

        pltpu.sync_copy(idx_hbm.at[pl.ds(wid * n_idx, n_idx)], idx_v)

        def gather_start(i, b):
            for g in range(G):
                pltpu.async_copy(table_hbm.at[idx_v.at[i * G + g]],
                                 rows[b].at[pl.ds(g * C, C)], gsem[b])

        def gather_wait(i, b):
            for g in range(G):
                pltpu.make_async_copy(
                    table_hbm.at[idx_v.at[i * G + g]],
                    rows[b].at[pl.ds(g * C, C)], gsem[b]).wait()

        def scatter_start(i, b):
            pltpu.async_copy(
                rows[b], out_hbm.at[pl.ds(base + i * CB, CB)], ssem[b])

        def scatter_wait(i, b):
            pltpu.make_async_copy(
                rows[b], out_hbm.at[pl.ds(base + i * CB, CB)], ssem[b]).wait()

        gather_start(0, 0)
        gather_start(1, 1)
        gather_wait(0, 0)
        scatter_start(0, 0)
        gather_wait(1, 1)
        scatter_start(1, 1)

        def body(j, carry):
            i0 = 2 * j
            i1 = i0 + 1
            scatter_wait(i0 - 2, 0)
            gather_start(i0, 0)
            scatter_wait(i1 - 2, 1)
            gather_start(i1, 1)
            gather_wait(i0, 0)
            scatter_start(i0, 0)
            gather_wait(i1, 1)
            scatter_start(i1, 1)
            return carry

        lax.fori_loop(1, n_chunks // 2, body, 0)

        scatter_wait(n_chunks - 2, 0)
        scatter_wait(n_chunks - 1, 1)

    return gather_kernel


def kernel(Tokens, table):
    S, T = Tokens.shape
    V, D = table.shape
    B = S * T
    scaled = _make_scale(V, D)(table)
    idx2d = Tokens.reshape(B // 128, 128).astype(jnp.int32)
    out = _make_gather(V, D, B)(idx2d, scaled)
    return out.reshape(S, T, D)


# speedup vs baseline: 9.0956x; 1.1649x over previous
import functools
import math

import jax
import jax.numpy as jnp
from jax import lax
from jax.experimental import pallas as pl
from jax.experimental.pallas import tpu as pltpu
from jax.experimental.pallas import tpu_sc as plsc

_D = 128
_SCALE = math.sqrt(float(_D))


def _scale_body(x_ref, o_ref):
    o_ref[...] = x_ref[...] * _SCALE


@functools.lru_cache(maxsize=None)
def _make_scale(V, D):
    blk = 2000
    assert V % blk == 0
    return pl.pallas_call(
        _scale_body,
        out_shape=jax.ShapeDtypeStruct((V, D), jnp.float32),
        grid=(V // blk,),
        in_specs=[pl.BlockSpec((blk, D), lambda i: (i, 0))],
        out_specs=pl.BlockSpec((blk, D), lambda i: (i, 0)),
    )


@functools.lru_cache(maxsize=None)
def _make_gather(V, D, B):
    info = plsc.get_sparse_core_info()
    NC, NS = info.num_cores, info.num_subcores
    NW = NC * NS
    C = 128
    G = 2
    CB = C * G
    assert B % (NW * CB) == 0
    b_per_w = B // NW
    n_idx = b_per_w // C
    n_chunks = b_per_w // CB
    mesh = plsc.VectorSubcoreMesh(core_axis_name="c", subcore_axis_name="s")

    @functools.partial(
        pl.kernel,
        out_type=jax.ShapeDtypeStruct((B, D), jnp.float32),
        mesh=mesh,
        scratch_types=[
            pltpu.VMEM((n_idx, C), jnp.int32),
            pltpu.VMEM((CB, D), jnp.float32),
            pltpu.VMEM((CB, D), jnp.float32),
            pltpu.SemaphoreType.DMA,
            pltpu.SemaphoreType.DMA,
            pltpu.SemaphoreType.DMA,
            pltpu.SemaphoreType.DMA,
        ],
    )
    def gather_kernel(idx_hbm, table_hbm, out_hbm,
                      idx_v, rows0, rows1, g0, g1, s0, s1):
        wid = lax.axis_index("s") * NC + lax.axis_index("c")
        base = wid * b_per_w
        rows = (rows0, rows1)
        gsem = (g0, g1)
        ssem = (s0, s1)

        pltpu.sync_copy(idx_hbm.at[pl.ds(wid * n_idx, n_idx)], idx_v)

        def gather_start(i, b):
            for g in range(G):
                pltpu.async_copy(table_hbm.at[idx_v.at[i * G + g]],
                                 rows[b].at[pl.ds(g * C, C)], gsem[b])

        def gather_wait(i, b):
            for g in range(G):
                pltpu.make_async_copy(
                    table_hbm.at[idx_v.at[i * G + g]],
                    rows[b].at[pl.ds(g * C, C)], gsem[b]).wait()

        def scale_buf(b):
            def sbody(r, carry):
                for u in range(2):
                    for k in range(D // 16):
                        sl = (2 * r + u, pl.ds(16 * k, 16))
                        rows[b][sl] = rows[b][sl] * _SCALE
                return carry
            lax.fori_loop(0, CB // 2, sbody, 0)

        def scatter_start(i, b):
            pltpu.async_copy(
                rows[b], out_hbm.at[pl.ds(base + i * CB, CB)], ssem[b])

        def scatter_wait(i, b):
            pltpu.make_async_copy(
                rows[b], out_hbm.at[pl.ds(base + i * CB, CB)], ssem[b]).wait()

        gather_start(0, 0)
        gather_start(1, 1)
        gather_wait(0, 0)
        scale_buf(0)
        scatter_start(0, 0)
        gather_wait(1, 1)
        scale_buf(1)
        scatter_start(1, 1)

        def body(j, carry):
            i0 = 2 * j
            i1 = i0 + 1
            scatter_wait(i0 - 2, 0)
            gather_start(i0, 0)
            scatter_wait(i1 - 2, 1)
            gather_start(i1, 1)
            gather_wait(i0, 0)
            scale_buf(0)
            scatter_start(i0, 0)
            gather_wait(i1, 1)
            scale_buf(1)
            scatter_start(i1, 1)
            return carry

        lax.fori_loop(1, n_chunks // 2, body, 0)

        scatter_wait(n_chunks - 2, 0)
        scatter_wait(n_chunks - 1, 1)

    return gather_kernel


def kernel(Tokens, table):
    S, T = Tokens.shape
    V, D = table.shape
    B = S * T
    idx2d = Tokens.reshape(B // 128, 128).astype(jnp.int32)
    out = _make_gather(V, D, B)(idx2d, table)
    return out.reshape(S, T, D)
